# baseline (speedup 1.0000x reference)
import functools

import jax
import jax.numpy as jnp
from jax import lax
from jax.experimental import pallas as pl
from jax.experimental.pallas import tpu as pltpu
from jax.experimental.pallas import tpu_sc as plsc

B = 16384
D = 32
L = 16
NC = 2
NS = 16
NV = 100000
NH = 1000000
CH = 16384
NSH = NH // CH
HREM = 640
HREM_B = NH - NSH * CH
VCH = [CH] * 6 + [1792]
VCH_B = [CH] * 6 + [NV - 6 * CH]
NBKT = 62
BSH = 14
NGRP = B // L

_mesh = plsc.VectorSubcoreMesh(core_axis_name="c", subcore_axis_name="s")


@functools.partial(
    pl.kernel,
    out_type=[jax.ShapeDtypeStruct((B,), jnp.float32),
              jax.ShapeDtypeStruct((B,), jnp.float32)],
    mesh=_mesh,
    compiler_params=pltpu.CompilerParams(
        needs_layout_passes=False, use_tc_tiling_on_sc=True),
    scratch_types=[
        pltpu.VMEM((B,), jnp.int32),
        pltpu.VMEM((B,), jnp.int32),
        pltpu.VMEM((B,), jnp.float32),
        pltpu.VMEM((1, CH), jnp.float32),
        pltpu.VMEM((CH,), jnp.float32),
        pltpu.VMEM((1, 128), jnp.int32),
        pltpu.VMEM((128,), jnp.float32),
        pltpu.VMEM((64, L), jnp.int32),
        pltpu.VMEM((64, L), jnp.int32),
        pltpu.SMEM((64,), jnp.int32),
        pltpu.VMEM_SHARED((2, 8, CH), jnp.float32),
        pltpu.VMEM_SHARED((B,), jnp.float32),
        pltpu.SemaphoreType.DMA,
    ],
)
def _gmf_sc(v_idx_hbm, h_idx_hbm, virus_t, human_t, vb_hbm, hb_hbm,
            out0_hbm, out1_hbm, idx_h, idx_v, vvals, plane, bias_pl,
            ids, cvals, hist, nxt, bstart, slab, acc_sh, sem):
    cid = lax.axis_index("c")
    sid = lax.axis_index("s")
    olocal = sid // 8
    j = sid % 8
    lanes = lax.iota(jnp.int32, L)
    zero16 = jnp.zeros((L,), jnp.int32)

    pltpu.sync_copy(h_idx_hbm, idx_h)
    pltpu.sync_copy(v_idx_hbm, idx_v)

    is_vb = (cid == 0) & (sid == 0)
    is_hb = (cid == 1) & (sid == 0)

    @pl.when(sid == 0)
    def _():
        for k in range(8):
            cvals[pl.ds(k * L, L)] = jnp.zeros((L,), jnp.float32)
        for k in range(B // 128):
            pltpu.sync_copy(cvals, acc_sh.at[pl.ds(k * 128, 128)])

    row0 = pl.multiple_of(cid * 16, 8)

    def scan_chunk(c0, cw, first):
        def body(g, carry):
            s = pl.multiple_of(g * L, L)
            v = idx_v[pl.ds(s, L)]
            m = (v >= c0) & (v < c0 + cw)
            col = jnp.clip(v - c0, 0, cw - 1)
            x = plsc.load_gather(plane, [zero16, col])
            contrib = jnp.where(m, x, 0.0)
            if first:
                vvals[pl.ds(s, L)] = contrib
            else:
                vvals[pl.ds(s, L)] = vvals[pl.ds(s, L)] + contrib
            return carry
        lax.fori_loop(0, NGRP, body, 0)

    for vc in range(7):
        c0 = vc * CH
        cw = VCH[vc]
        c0d = pl.multiple_of(jnp.int32(c0), CH) if vc == 6 else c0

        @pl.when(sid == 0)
        def _(c0d=c0d, cw=cw):
            pltpu.sync_copy(
                virus_t.at[pl.ds(row0, 8), pl.ds(c0d, cw)],
                slab.at[0].at[:, pl.ds(0, cw)])
            pltpu.sync_copy(
                virus_t.at[pl.ds(row0 + 8, 8), pl.ds(c0d, cw)],
                slab.at[1].at[:, pl.ds(0, cw)])
        plsc.subcore_barrier()

        @pl.when(olocal == 0)
        def _():
            pltpu.sync_copy(slab.at[0].at[pl.ds(j, 1), pl.ds(0, cw)],
                            plane.at[:, pl.ds(0, cw)])

        @pl.when(olocal == 1)
        def _():
            pltpu.sync_copy(slab.at[1].at[pl.ds(j, 1), pl.ds(0, cw)],
                            plane.at[:, pl.ds(0, cw)])
        scan_chunk(c0, cw, vc == 0)

        cwb = VCH_B[vc]

        @pl.when(is_vb)
        def _(c0=c0, cw=cwb, vc=vc):
            pltpu.sync_copy(vb_hbm.at[pl.ds(c0, cw)],
                            bias_pl.at[pl.ds(0, cw)])
            def bbody(g, carry):
                s = pl.multiple_of(g * L, L)
                v = idx_v[pl.ds(s, L)]
                m = (v >= c0) & (v < c0 + cw)
                col = jnp.clip(v - c0, 0, cw - 1)
                xb = plsc.load_gather(bias_pl, [col])
                contrib = jnp.where(m, xb, 0.0)
                ko = pl.multiple_of((g % 8) * L, L)
                ids[0, pl.ds(ko, L)] = s + lanes
                cvals[pl.ds(ko, L)] = contrib
                return carry

            def wave(w, carry):
                lax.fori_loop(w * 8, w * 8 + 8, bbody, 0)
                pltpu.sync_copy(cvals, acc_sh.at[ids.at[0]], add=True)
                return carry
            lax.fori_loop(0, NGRP // 8, wave, 0)
        plsc.subcore_barrier()

    ones16 = jnp.ones((L,), jnp.int32)
    for b in range(64):
        hist[b, pl.ds(0, L)] = zero16

    def h1(g, carry):
        s = pl.multiple_of(g * L, L)
        bins = lax.shift_right_logical(idx_h[pl.ds(s, L)], BSH)
        plsc.addupdate_scatter(hist, [bins, lanes], ones16)
        return carry
    lax.fori_loop(0, NGRP, h1, 0)

    run = jnp.int32(0)
    for b in range(NBKT):
        brow = plsc.load_gather(hist, [jnp.full((L,), b, jnp.int32), lanes])
        bstart[b] = run
        lane_pref = plsc.cumsum(brow) - brow + run
        plsc.store_scatter(nxt, [jnp.full((L,), b, jnp.int32), lanes],
                           lane_pref)
        run = run + lax.reduce_sum(brow, (0,))
    bstart[NBKT] = run

    def h2(g, carry):
        s = pl.multiple_of(g * L, L)
        bins = lax.shift_right_logical(idx_h[pl.ds(s, L)], BSH)
        pos = plsc.load_gather(nxt, [bins, lanes])
        plsc.store_scatter(nxt, [bins, lanes], pos + 1)
        plsc.store_scatter(idx_v, [pos], s + lanes)
        return carry
    lax.fori_loop(0, NGRP, h2, 0)
    perm = idx_v

    def do_slab(s_idx, c0, cw, cwb):
        @pl.when(sid == 0)
        def _():
            pltpu.sync_copy(
                human_t.at[pl.ds(row0, 8), pl.ds(c0, cw)],
                slab.at[0].at[:, pl.ds(0, cw)])
            pltpu.sync_copy(
                human_t.at[pl.ds(row0 + 8, 8), pl.ds(c0, cw)],
                slab.at[1].at[:, pl.ds(0, cw)])
        plsc.subcore_barrier()

        @pl.when(olocal == 0)
        def _():
            pltpu.sync_copy(slab.at[0].at[pl.ds(j, 1), pl.ds(0, cw)],
                            plane.at[:, pl.ds(0, cw)])

        @pl.when(olocal == 1)
        def _():
            pltpu.sync_copy(slab.at[1].at[pl.ds(j, 1), pl.ds(0, cw)],
                            plane.at[:, pl.ds(0, cw)])

        @pl.when(is_hb)
        def _():
            pltpu.sync_copy(hb_hbm.at[pl.ds(c0, cwb)],
                            bias_pl.at[pl.ds(0, cwb)])

        start = bstart[s_idx]
        end = bstart[s_idx + 1]

        def gbody(k, carry):
            p = start + k * L + lanes
            pm = p < end
            e = plsc.load_gather(perm, [jnp.clip(p, 0, B - 1)])
            h = plsc.load_gather(idx_h, [e])
            col = jnp.clip(h - c0, 0, cw - 1)
            x = plsc.load_gather(plane, [zero16, col])
            vv = plsc.load_gather(vvals, [e])
            contrib = x * vv
            contrib = contrib + jnp.where(
                is_hb, plsc.load_gather(bias_pl, [col]), 0.0)
            contrib = jnp.where(pm, contrib, 0.0)
            ko = pl.multiple_of((k % 8) * L, L)
            ids[0, pl.ds(ko, L)] = e
            cvals[pl.ds(ko, L)] = contrib
            return carry

        def wave(w, carry):
            lax.fori_loop(w * 8, w * 8 + 8, gbody, 0)
            pltpu.sync_copy(cvals, acc_sh.at[ids.at[0]], add=True)
            return carry
        nwaves = (end - start + 127) // 128
        lax.fori_loop(0, nwaves, wave, 0)
        plsc.subcore_barrier()

    def slab_loop(s_idx, carry):
        c0 = pl.multiple_of(s_idx * CH, CH)
        do_slab(s_idx, c0, CH, CH)
        return carry
    lax.fori_loop(0, NSH, slab_loop, 0)
    do_slab(jnp.int32(NSH), pl.multiple_of(jnp.int32(NSH * CH), CH),
            HREM, HREM_B)

    @pl.when(sid == 0)
    def _():
        @pl.when(cid == 0)
        def _():
            pltpu.sync_copy(acc_sh, out0_hbm)

        @pl.when(cid == 1)
        def _():
            pltpu.sync_copy(acc_sh, out1_hbm)


def kernel(v_idxs, h_idxs, virus, human, virus_b, human_b):
    o0, o1 = _gmf_sc(v_idxs, h_idxs, virus.T, human.T,
                     virus_b.reshape(-1), human_b.reshape(-1))
    return o0 + o1


# speedup vs baseline: 1.4252x; 1.4252x over previous
import functools

import jax
import jax.numpy as jnp
from jax import lax
from jax.experimental import pallas as pl
from jax.experimental.pallas import tpu as pltpu
from jax.experimental.pallas import tpu_sc as plsc

B = 16384
D = 32
L = 16
NC = 2
NS = 16
NV = 100000
NH = 1000000
CH = 16384
NSH = NH // CH
HREM = 640
HREM_B = NH - NSH * CH
VCH = [CH] * 6 + [1792]
VCH_B = [CH] * 6 + [NV - 6 * CH]
NBKT = 62
BSH = 14
NGRP = B // L

_mesh = plsc.VectorSubcoreMesh(core_axis_name="c", subcore_axis_name="s")


@functools.partial(
    pl.kernel,
    out_type=[jax.ShapeDtypeStruct((B,), jnp.float32),
              jax.ShapeDtypeStruct((B,), jnp.float32)],
    mesh=_mesh,
    compiler_params=pltpu.CompilerParams(
        needs_layout_passes=False, use_tc_tiling_on_sc=True),
    scratch_types=[
        pltpu.VMEM((B,), jnp.int32),
        pltpu.VMEM((B,), jnp.int32),
        pltpu.VMEM((B,), jnp.float32),
        pltpu.VMEM((1, CH), jnp.float32),
        pltpu.VMEM((CH,), jnp.float32),
        pltpu.VMEM((1, B), jnp.float32),
        pltpu.VMEM((64, L), jnp.int32),
        pltpu.VMEM((64, L), jnp.int32),
        pltpu.SMEM((64,), jnp.int32),
        pltpu.VMEM_SHARED((2, 8, CH), jnp.float32),
        pltpu.SemaphoreType.DMA,
    ],
)
def _gmf_sc(v_idx_hbm, h_idx_hbm, virus_t, human_t, vb_hbm, hb_hbm,
            out0_hbm, out1_hbm, idx_h, idx_v, vvals, plane, bias_pl,
            acc_l, hist, nxt, bstart, slab, sem):
    cid = lax.axis_index("c")
    sid = lax.axis_index("s")
    olocal = sid // 8
    j = sid % 8
    lanes = lax.iota(jnp.int32, L)
    zero16 = jnp.zeros((L,), jnp.int32)

    pltpu.sync_copy(h_idx_hbm, idx_h)
    pltpu.sync_copy(v_idx_hbm, idx_v)

    is_vb = (cid == 0) & (sid == 0)
    is_hb = (cid == 1) & (sid == 0)

    zf = jnp.zeros((L,), jnp.float32)

    def zbody(g, carry):
        s = pl.multiple_of(g * 4 * L, L)
        for k in range(4):
            acc_l[0, pl.ds(s + k * L, L)] = zf
        return carry
    lax.fori_loop(0, NGRP // 4, zbody, 0)

    row0 = pl.multiple_of(cid * 16, 8)

    def scan_chunk(c0, cw, first):
        def body(g, carry):
            for k in range(4):
                s = pl.multiple_of(g * 4 * L + k * L, L)
                v = idx_v[pl.ds(s, L)]
                m = (v >= c0) & (v < c0 + cw)
                col = jnp.clip(v - c0, 0, cw - 1)
                x = plsc.load_gather(plane, [zero16, col])
                contrib = jnp.where(m, x, 0.0)
                if first:
                    vvals[pl.ds(s, L)] = contrib
                else:
                    vvals[pl.ds(s, L)] = vvals[pl.ds(s, L)] + contrib
            return carry
        lax.fori_loop(0, NGRP // 4, body, 0)

    for vc in range(7):
        c0 = vc * CH
        cw = VCH[vc]
        c0d = pl.multiple_of(jnp.int32(c0), CH) if vc == 6 else c0

        @pl.when(sid == 0)
        def _(c0d=c0d, cw=cw):
            ca = pltpu.async_copy(
                virus_t.at[pl.ds(row0, 8), pl.ds(c0d, cw)],
                slab.at[0].at[:, pl.ds(0, cw)], sem)
            cb = pltpu.async_copy(
                virus_t.at[pl.ds(row0 + 8, 8), pl.ds(c0d, cw)],
                slab.at[1].at[:, pl.ds(0, cw)], sem)
            ca.wait()
            cb.wait()
        plsc.subcore_barrier()

        @pl.when(olocal == 0)
        def _():
            pltpu.sync_copy(slab.at[0].at[pl.ds(j, 1), pl.ds(0, cw)],
                            plane.at[:, pl.ds(0, cw)])

        @pl.when(olocal == 1)
        def _():
            pltpu.sync_copy(slab.at[1].at[pl.ds(j, 1), pl.ds(0, cw)],
                            plane.at[:, pl.ds(0, cw)])
        scan_chunk(c0, cw, vc == 0)

        cwb = VCH_B[vc]

        @pl.when(is_vb)
        def _(c0=c0, cw=cwb, vc=vc):
            pltpu.sync_copy(vb_hbm.at[pl.ds(c0, cw)],
                            bias_pl.at[pl.ds(0, cw)])
            def bbody(g, carry):
                s = pl.multiple_of(g * L, L)
                v = idx_v[pl.ds(s, L)]
                m = (v >= c0) & (v < c0 + cw)
                col = jnp.clip(v - c0, 0, cw - 1)
                xb = plsc.load_gather(bias_pl, [col])
                contrib = jnp.where(m, xb, 0.0)
                acc_l[0, pl.ds(s, L)] = acc_l[0, pl.ds(s, L)] + contrib
                return carry
            lax.fori_loop(0, NGRP, bbody, 0)
        plsc.subcore_barrier()

    ones16 = jnp.ones((L,), jnp.int32)
    for b in range(64):
        hist[b, pl.ds(0, L)] = zero16

    def h1(g, carry):
        s = pl.multiple_of(g * L, L)
        bins = lax.shift_right_logical(idx_h[pl.ds(s, L)], BSH)
        plsc.addupdate_scatter(hist, [bins, lanes], ones16)
        return carry
    lax.fori_loop(0, NGRP, h1, 0)

    run = jnp.int32(0)
    for b in range(NBKT):
        brow = plsc.load_gather(hist, [jnp.full((L,), b, jnp.int32), lanes])
        bstart[b] = run
        lane_pref = plsc.cumsum(brow) - brow + run
        plsc.store_scatter(nxt, [jnp.full((L,), b, jnp.int32), lanes],
                           lane_pref)
        run = run + lax.reduce_sum(brow, (0,))
    bstart[NBKT] = run

    def h2(g, carry):
        s = pl.multiple_of(g * L, L)
        bins = lax.shift_right_logical(idx_h[pl.ds(s, L)], BSH)
        pos = plsc.load_gather(nxt, [bins, lanes])
        plsc.store_scatter(nxt, [bins, lanes], pos + 1)
        plsc.store_scatter(idx_v, [pos], s + lanes)
        return carry
    lax.fori_loop(0, NGRP, h2, 0)
    perm = idx_v

    def do_slab(s_idx, c0, cw, cwb):
        @pl.when(sid == 0)
        def _():
            ca = pltpu.async_copy(
                human_t.at[pl.ds(row0, 8), pl.ds(c0, cw)],
                slab.at[0].at[:, pl.ds(0, cw)], sem)
            cb = pltpu.async_copy(
                human_t.at[pl.ds(row0 + 8, 8), pl.ds(c0, cw)],
                slab.at[1].at[:, pl.ds(0, cw)], sem)
            ca.wait()
            cb.wait()
        plsc.subcore_barrier()

        @pl.when(olocal == 0)
        def _():
            pltpu.sync_copy(slab.at[0].at[pl.ds(j, 1), pl.ds(0, cw)],
                            plane.at[:, pl.ds(0, cw)])

        @pl.when(olocal == 1)
        def _():
            pltpu.sync_copy(slab.at[1].at[pl.ds(j, 1), pl.ds(0, cw)],
                            plane.at[:, pl.ds(0, cw)])

        @pl.when(is_hb)
        def _():
            pltpu.sync_copy(hb_hbm.at[pl.ds(c0, cwb)],
                            bias_pl.at[pl.ds(0, cwb)])

        start = bstart[s_idx]
        end = bstart[s_idx + 1]

        def gbody(k, carry):
            p = start + k * L + lanes
            pm = p < end
            e = plsc.load_gather(perm, [jnp.clip(p, 0, B - 1)])
            h = plsc.load_gather(idx_h, [e])
            col = jnp.clip(h - c0, 0, cw - 1)
            x = plsc.load_gather(plane, [zero16, col])
            vv = plsc.load_gather(vvals, [e])
            contrib = x * vv
            contrib = contrib + jnp.where(
                is_hb, plsc.load_gather(bias_pl, [col]), 0.0)
            contrib = jnp.where(pm, contrib, 0.0)
            cur = plsc.load_gather(acc_l, [zero16, e])
            plsc.store_scatter(acc_l, [zero16, e], cur + contrib, mask=pm)
            return carry

        ngroups = (end - start + L - 1) // L
        lax.fori_loop(0, ngroups, gbody, 0)
        plsc.subcore_barrier()

    def slab_loop(s_idx, carry):
        c0 = pl.multiple_of(s_idx * CH, CH)
        do_slab(s_idx, c0, CH, CH)
        return carry
    lax.fori_loop(0, NSH, slab_loop, 0)
    do_slab(jnp.int32(NSH), pl.multiple_of(jnp.int32(NSH * CH), CH),
            HREM, HREM_B)

    def tree_add():
        def add_from_plane(g, carry):
            for k in range(4):
                st = pl.multiple_of(g * 4 * L + k * L, L)
                acc_l[0, pl.ds(st, L)] = (
                    acc_l[0, pl.ds(st, L)] + plane[0, pl.ds(st, L)])
            return carry
        lax.fori_loop(0, NGRP // 4, add_from_plane, 0)

    @pl.when(olocal == 0)
    def _():
        pltpu.sync_copy(acc_l, slab.at[0].at[pl.ds(j, 1), :])

    @pl.when(olocal == 1)
    def _():
        pltpu.sync_copy(acc_l, slab.at[1].at[pl.ds(j, 1), :])
    plsc.subcore_barrier()

    @pl.when(sid < 2)
    def _():
        r0 = sid * 4
        pltpu.sync_copy(slab.at[0].at[pl.ds(r0, 1), :], acc_l)
        for i in range(1, 4):
            pltpu.sync_copy(slab.at[0].at[pl.ds(r0 + i, 1), :], plane)
            tree_add()

    @pl.when((sid >= 2) & (sid < 4))
    def _():
        r0 = sid * 4 - 8
        pltpu.sync_copy(slab.at[1].at[pl.ds(r0, 1), :], acc_l)
        for i in range(1, 4):
            pltpu.sync_copy(slab.at[1].at[pl.ds(r0 + i, 1), :], plane)
            tree_add()
    plsc.subcore_barrier()

    @pl.when((sid >= 1) & (sid < 4))
    def _():
        pltpu.sync_copy(acc_l, slab.at[0].at[pl.ds(sid - 1, 1), :])
    plsc.subcore_barrier()

    @pl.when(sid == 0)
    def _():
        for k in range(3):
            pltpu.sync_copy(slab.at[0].at[pl.ds(k, 1), :], plane)
            tree_add()

    @pl.when(sid == 0)
    def _():
        @pl.when(cid == 0)
        def _():
            pltpu.sync_copy(acc_l.at[0], out0_hbm)

        @pl.when(cid == 1)
        def _():
            pltpu.sync_copy(acc_l.at[0], out1_hbm)


def kernel(v_idxs, h_idxs, virus, human, virus_b, human_b):
    o0, o1 = _gmf_sc(v_idxs, h_idxs, virus.T, human.T,
                     virus_b.reshape(-1), human_b.reshape(-1))
    return o0 + o1


# speedup vs baseline: 1.4279x; 1.0019x over previous
import functools

import jax
import jax.numpy as jnp
from jax import lax
from jax.experimental import pallas as pl
from jax.experimental.pallas import tpu as pltpu
from jax.experimental.pallas import tpu_sc as plsc

B = 16384
D = 32
L = 16
NC = 2
NS = 16
NV = 100000
NH = 1000000
CH = 16384
NSH = NH // CH
HREM = 640
HREM_B = NH - NSH * CH
VCH = [CH] * 6 + [1792]
VCH_B = [CH] * 6 + [NV - 6 * CH]
NBKT = 62
BSH = 14
NGRP = B // L

_mesh = plsc.VectorSubcoreMesh(core_axis_name="c", subcore_axis_name="s")


@functools.partial(
    pl.kernel,
    out_type=[jax.ShapeDtypeStruct((B,), jnp.float32),
              jax.ShapeDtypeStruct((B,), jnp.float32)],
    mesh=_mesh,
    compiler_params=pltpu.CompilerParams(
        needs_layout_passes=False, use_tc_tiling_on_sc=True),
    scratch_types=[
        pltpu.VMEM((B,), jnp.int32),
        pltpu.VMEM((B,), jnp.int32),
        pltpu.VMEM((B,), jnp.float32),
        pltpu.VMEM((1, CH), jnp.float32),
        pltpu.VMEM((CH,), jnp.float32),
        pltpu.VMEM((1, B), jnp.float32),
        pltpu.VMEM((64, L), jnp.int32),
        pltpu.VMEM((64, L), jnp.int32),
        pltpu.SMEM((64,), jnp.int32),
        pltpu.VMEM_SHARED((2, 8, CH), jnp.float32),
        pltpu.SemaphoreType.DMA,
    ],
)
def _gmf_sc(v_idx_hbm, h_idx_hbm, virus_t, human_t, vb_hbm, hb_hbm,
            out0_hbm, out1_hbm, idx_h, idx_v, vvals, plane, bias_pl,
            acc_l, hist, nxt, bstart, slab, sem):
    cid = lax.axis_index("c")
    sid = lax.axis_index("s")
    olocal = sid // 8
    j = sid % 8
    lanes = lax.iota(jnp.int32, L)
    zero16 = jnp.zeros((L,), jnp.int32)

    pltpu.sync_copy(h_idx_hbm, idx_h)
    pltpu.sync_copy(v_idx_hbm, idx_v)

    is_vb = (cid == 0) & (sid == 0)
    is_hb = (cid == 1) & (sid == 0)

    zf = jnp.zeros((L,), jnp.float32)

    def zbody(g, carry):
        s = pl.multiple_of(g * 4 * L, L)
        for k in range(4):
            acc_l[0, pl.ds(s + k * L, L)] = zf
        return carry
    lax.fori_loop(0, NGRP // 4, zbody, 0)

    row0 = pl.multiple_of(cid * 16, 8)

    def scan_chunk(c0, cw, first):
        def body(g, carry):
            for k in range(4):
                s = pl.multiple_of(g * 4 * L + k * L, L)
                v = idx_v[pl.ds(s, L)]
                m = (v >= c0) & (v < c0 + cw)
                col = jnp.clip(v - c0, 0, cw - 1)
                x = plsc.load_gather(plane, [zero16, col])
                contrib = jnp.where(m, x, 0.0)
                if first:
                    vvals[pl.ds(s, L)] = contrib
                else:
                    vvals[pl.ds(s, L)] = vvals[pl.ds(s, L)] + contrib
            return carry
        lax.fori_loop(0, NGRP // 4, body, 0)

    for vc in range(7):
        c0 = vc * CH
        cw = VCH[vc]
        c0d = pl.multiple_of(jnp.int32(c0), CH) if vc == 6 else c0

        @pl.when(sid == 0)
        def _(c0d=c0d, cw=cw):
            ca = pltpu.async_copy(
                virus_t.at[pl.ds(row0, 8), pl.ds(c0d, cw)],
                slab.at[0].at[:, pl.ds(0, cw)], sem)
            cb = pltpu.async_copy(
                virus_t.at[pl.ds(row0 + 8, 8), pl.ds(c0d, cw)],
                slab.at[1].at[:, pl.ds(0, cw)], sem)
            ca.wait()
            cb.wait()
        plsc.subcore_barrier()

        @pl.when(olocal == 0)
        def _():
            pltpu.sync_copy(slab.at[0].at[pl.ds(j, 1), pl.ds(0, cw)],
                            plane.at[:, pl.ds(0, cw)])

        @pl.when(olocal == 1)
        def _():
            pltpu.sync_copy(slab.at[1].at[pl.ds(j, 1), pl.ds(0, cw)],
                            plane.at[:, pl.ds(0, cw)])
        scan_chunk(c0, cw, vc == 0)

        cwb = VCH_B[vc]

        @pl.when(is_vb)
        def _(c0=c0, cw=cwb, vc=vc):
            pltpu.sync_copy(vb_hbm.at[pl.ds(c0, cw)],
                            bias_pl.at[pl.ds(0, cw)])
            def bbody(g, carry):
                s = pl.multiple_of(g * L, L)
                v = idx_v[pl.ds(s, L)]
                m = (v >= c0) & (v < c0 + cw)
                col = jnp.clip(v - c0, 0, cw - 1)
                xb = plsc.load_gather(bias_pl, [col])
                contrib = jnp.where(m, xb, 0.0)
                acc_l[0, pl.ds(s, L)] = acc_l[0, pl.ds(s, L)] + contrib
                return carry
            lax.fori_loop(0, NGRP, bbody, 0)
        plsc.subcore_barrier()

    ones16 = jnp.ones((L,), jnp.int32)
    for b in range(64):
        hist[b, pl.ds(0, L)] = zero16

    def h1(g, carry):
        s = pl.multiple_of(g * L, L)
        bins = lax.shift_right_logical(idx_h[pl.ds(s, L)], BSH)
        plsc.addupdate_scatter(hist, [bins, lanes], ones16)
        return carry
    lax.fori_loop(0, NGRP, h1, 0)

    run = jnp.int32(0)
    for b in range(NBKT):
        brow = plsc.load_gather(hist, [jnp.full((L,), b, jnp.int32), lanes])
        bstart[b] = run
        lane_pref = plsc.cumsum(brow) - brow + run
        plsc.store_scatter(nxt, [jnp.full((L,), b, jnp.int32), lanes],
                           lane_pref)
        run = run + lax.reduce_sum(brow, (0,))
    bstart[NBKT] = run

    def h2(g, carry):
        s = pl.multiple_of(g * L, L)
        bins = lax.shift_right_logical(idx_h[pl.ds(s, L)], BSH)
        pos = plsc.load_gather(nxt, [bins, lanes])
        plsc.store_scatter(nxt, [bins, lanes], pos + 1)
        plsc.store_scatter(idx_v, [pos], s + lanes)
        return carry
    lax.fori_loop(0, NGRP, h2, 0)
    perm = idx_v

    def do_slab(s_idx, c0, cw, cwb):
        @pl.when(sid == 0)
        def _():
            ca = pltpu.async_copy(
                human_t.at[pl.ds(row0, 8), pl.ds(c0, cw)],
                slab.at[0].at[:, pl.ds(0, cw)], sem)
            cb = pltpu.async_copy(
                human_t.at[pl.ds(row0 + 8, 8), pl.ds(c0, cw)],
                slab.at[1].at[:, pl.ds(0, cw)], sem)
            ca.wait()
            cb.wait()
        plsc.subcore_barrier()

        @pl.when(olocal == 0)
        def _():
            pltpu.sync_copy(slab.at[0].at[pl.ds(j, 1), pl.ds(0, cw)],
                            plane.at[:, pl.ds(0, cw)])

        @pl.when(olocal == 1)
        def _():
            pltpu.sync_copy(slab.at[1].at[pl.ds(j, 1), pl.ds(0, cw)],
                            plane.at[:, pl.ds(0, cw)])

        @pl.when(is_hb)
        def _():
            pltpu.sync_copy(hb_hbm.at[pl.ds(c0, cwb)],
                            bias_pl.at[pl.ds(0, cwb)])

        start = bstart[s_idx]
        end = bstart[s_idx + 1]

        def gbody(k, carry):
            p = start + k * L + lanes
            pm = p < end
            e = plsc.load_gather(perm, [jnp.clip(p, 0, B - 1)])
            h = plsc.load_gather(idx_h, [e])
            col = jnp.clip(h - c0, 0, cw - 1)
            x = plsc.load_gather(plane, [zero16, col])
            vv = plsc.load_gather(vvals, [e])
            contrib = x * vv
            contrib = contrib + jnp.where(
                is_hb, plsc.load_gather(bias_pl, [col]), 0.0)
            contrib = jnp.where(pm, contrib, 0.0)
            cur = plsc.load_gather(acc_l, [zero16, e])
            plsc.store_scatter(acc_l, [zero16, e], cur + contrib, mask=pm)
            return carry

        plsc.subcore_barrier()
        ngroups = (end - start + L - 1) // L
        lax.fori_loop(0, ngroups, gbody, 0)

    def slab_loop(s_idx, carry):
        c0 = pl.multiple_of(s_idx * CH, CH)
        do_slab(s_idx, c0, CH, CH)
        return carry
    lax.fori_loop(0, NSH, slab_loop, 0)
    do_slab(jnp.int32(NSH), pl.multiple_of(jnp.int32(NSH * CH), CH),
            HREM, HREM_B)

    def tree_add():
        def add_from_plane(g, carry):
            for k in range(4):
                st = pl.multiple_of(g * 4 * L + k * L, L)
                acc_l[0, pl.ds(st, L)] = (
                    acc_l[0, pl.ds(st, L)] + plane[0, pl.ds(st, L)])
            return carry
        lax.fori_loop(0, NGRP // 4, add_from_plane, 0)

    @pl.when(olocal == 0)
    def _():
        pltpu.sync_copy(acc_l, slab.at[0].at[pl.ds(j, 1), :])

    @pl.when(olocal == 1)
    def _():
        pltpu.sync_copy(acc_l, slab.at[1].at[pl.ds(j, 1), :])
    plsc.subcore_barrier()

    @pl.when(sid < 2)
    def _():
        r0 = sid * 4
        pltpu.sync_copy(slab.at[0].at[pl.ds(r0, 1), :], acc_l)
        for i in range(1, 4):
            pltpu.sync_copy(slab.at[0].at[pl.ds(r0 + i, 1), :], plane)
            tree_add()

    @pl.when((sid >= 2) & (sid < 4))
    def _():
        r0 = sid * 4 - 8
        pltpu.sync_copy(slab.at[1].at[pl.ds(r0, 1), :], acc_l)
        for i in range(1, 4):
            pltpu.sync_copy(slab.at[1].at[pl.ds(r0 + i, 1), :], plane)
            tree_add()
    plsc.subcore_barrier()

    @pl.when((sid >= 1) & (sid < 4))
    def _():
        pltpu.sync_copy(acc_l, slab.at[0].at[pl.ds(sid - 1, 1), :])
    plsc.subcore_barrier()

    @pl.when(sid == 0)
    def _():
        for k in range(3):
            pltpu.sync_copy(slab.at[0].at[pl.ds(k, 1), :], plane)
            tree_add()

    @pl.when(sid == 0)
    def _():
        @pl.when(cid == 0)
        def _():
            pltpu.sync_copy(acc_l.at[0], out0_hbm)

        @pl.when(cid == 1)
        def _():
            pltpu.sync_copy(acc_l.at[0], out1_hbm)


def kernel(v_idxs, h_idxs, virus, human, virus_b, human_b):
    o0, o1 = _gmf_sc(v_idxs, h_idxs, virus.T, human.T,
                     virus_b.reshape(-1), human_b.reshape(-1))
    return o0 + o1
